# baseline (speedup 1.0000x reference)
import functools

import jax
import jax.numpy as jnp
from jax import lax
from jax.experimental import pallas as pl
from jax.experimental.pallas import tpu as pltpu
from jax.experimental.pallas import tpu_sc as plsc

NC = 2
NS = 16
NW = NC * NS

CHUNK = 128


def _sc_gather(table, idx2d, *, cpw, d):
    n_rows = idx2d.shape[0] * CHUNK
    mesh = plsc.VectorSubcoreMesh(
        core_axis_name="c", subcore_axis_name="s", num_cores=NC, num_subcores=NS
    )

    @functools.partial(
        pl.kernel,
        out_type=jax.ShapeDtypeStruct((n_rows, d), jnp.float32),
        mesh=mesh,
        compiler_params=pltpu.CompilerParams(use_tc_tiling_on_sc=False),
        scratch_types=[
            pltpu.VMEM((cpw, CHUNK), jnp.int32),
            pltpu.VMEM((CHUNK, d), jnp.float32),
            pltpu.VMEM((CHUNK, d), jnp.float32),
            pltpu.SemaphoreType.DMA,
            pltpu.SemaphoreType.DMA,
        ],
    )
    def grab(table_hbm, idx_hbm, out_hbm, idx_v, buf0, buf1, sem0, sem1):
        wid = lax.axis_index("s") * NC + lax.axis_index("c")
        base = wid * cpw
        pltpu.sync_copy(idx_hbm.at[pl.ds(base, cpw)], idx_v)

        bufs = (buf0, buf1)
        sems = (sem0, sem1)

        def start(j, b):
            pltpu.make_async_copy(
                table_hbm.at[idx_v.at[j]], bufs[b], sems[b]
            ).start()

        def wait(b):
            pltpu.make_async_copy(
                table_hbm.at[idx_v.at[0]], bufs[b], sems[b]
            ).wait()

        start(0, 0)

        def outer(g, carry):
            for b in range(2):
                j = g * 2 + b

                @pl.when(j + 1 < cpw)
                def _():
                    start(j + 1, 1 - b)

                wait(b)
                pltpu.sync_copy(
                    bufs[b], out_hbm.at[pl.ds((base + j) * CHUNK, CHUNK)]
                )
            return carry

        lax.fori_loop(0, cpw // 2, outer, 0, unroll=False)

    return grab(table, idx2d)


def kernel(arg0_1, arg1_1):
    b0, b1 = arg0_1.shape
    v, d = arg1_1.shape
    n = b0 * b1
    n_chunks = n // CHUNK
    cpw = n_chunks // NW
    idx2d = arg0_1.astype(jnp.int32).reshape(n_chunks, CHUNK)
    rows = _sc_gather(arg1_1, idx2d, cpw=cpw, d=d)
    return (rows.reshape(b0, b1, d),)


# speedup vs baseline: 1.0105x; 1.0105x over previous
import functools

import jax
import jax.numpy as jnp
from jax import lax
from jax.experimental import pallas as pl
from jax.experimental.pallas import tpu as pltpu
from jax.experimental.pallas import tpu_sc as plsc

NC = 2
NS = 16
NW = NC * NS

SLABS = 8


def _sc_gather(table, idx2d, *, b0, b1, d):
    spw = b0 // NW
    nch = spw // SLABS
    mesh = plsc.VectorSubcoreMesh(
        core_axis_name="c", subcore_axis_name="s", num_cores=NC, num_subcores=NS
    )

    @functools.partial(
        pl.kernel,
        out_type=jax.ShapeDtypeStruct((b0, b1, d), jnp.float32),
        mesh=mesh,
        compiler_params=pltpu.CompilerParams(use_tc_tiling_on_sc=False),
        scratch_types=[
            pltpu.VMEM((spw, b1), jnp.int32),
            pltpu.VMEM((SLABS, b1, d), jnp.float32),
            pltpu.VMEM((SLABS, b1, d), jnp.float32),
            pltpu.SemaphoreType.DMA,
            pltpu.SemaphoreType.DMA,
        ],
    )
    def grab(table_hbm, idx_hbm, out_hbm, idx_v, buf0, buf1, sem0, sem1):
        wid = lax.axis_index("s") * NC + lax.axis_index("c")
        base = wid * spw
        pltpu.sync_copy(idx_hbm.at[pl.ds(base, spw)], idx_v)

        bufs = (buf0, buf1)
        sems = (sem0, sem1)

        def start(j, b):
            for g in range(SLABS):
                pltpu.make_async_copy(
                    table_hbm.at[idx_v.at[j * SLABS + g]], bufs[b].at[g], sems[b]
                ).start()

        def wait(b):
            for g in range(SLABS):
                pltpu.make_async_copy(
                    table_hbm.at[idx_v.at[0]], bufs[b].at[g], sems[b]
                ).wait()

        start(0, 0)

        def outer(g, carry):
            for b in range(2):
                j = g * 2 + b

                @pl.when(j + 1 < nch)
                def _():
                    start(j + 1, 1 - b)

                wait(b)
                pltpu.sync_copy(
                    bufs[b], out_hbm.at[pl.ds(base + j * SLABS, SLABS)]
                )
            return carry

        lax.fori_loop(0, nch // 2, outer, 0, unroll=False)

    return grab(table, idx2d)


def kernel(arg0_1, arg1_1):
    b0, b1 = arg0_1.shape
    v, d = arg1_1.shape
    idx2d = arg0_1.astype(jnp.int32)
    return (_sc_gather(arg1_1, idx2d, b0=b0, b1=b1, d=d),)


# speedup vs baseline: 1.0118x; 1.0013x over previous
import functools

import jax
import jax.numpy as jnp
from jax import lax
from jax.experimental import pallas as pl
from jax.experimental.pallas import tpu as pltpu
from jax.experimental.pallas import tpu_sc as plsc

NC = 2
NS = 16
NW = NC * NS

SLABS = 16


def _sc_gather(table, idx2d, *, b0, b1, d):
    spw = b0 // NW
    nch = spw // SLABS
    mesh = plsc.VectorSubcoreMesh(
        core_axis_name="c", subcore_axis_name="s", num_cores=NC, num_subcores=NS
    )

    @functools.partial(
        pl.kernel,
        out_type=jax.ShapeDtypeStruct((b0, b1, d), jnp.float32),
        mesh=mesh,
        compiler_params=pltpu.CompilerParams(use_tc_tiling_on_sc=False),
        scratch_types=[
            pltpu.VMEM((spw, b1), jnp.int32),
            pltpu.VMEM((SLABS, b1, d), jnp.float32),
            pltpu.VMEM((SLABS, b1, d), jnp.float32),
            pltpu.SemaphoreType.DMA,
            pltpu.SemaphoreType.DMA,
        ],
    )
    def grab(table_hbm, idx_hbm, out_hbm, idx_v, buf0, buf1, sem0, sem1):
        wid = lax.axis_index("s") * NC + lax.axis_index("c")
        base = wid * spw
        pltpu.sync_copy(idx_hbm.at[pl.ds(base, spw)], idx_v)

        bufs = (buf0, buf1)
        sems = (sem0, sem1)

        def start(j, b):
            for g in range(SLABS):
                pltpu.make_async_copy(
                    table_hbm.at[idx_v.at[j * SLABS + g]], bufs[b].at[g], sems[b]
                ).start()

        def wait(b):
            for g in range(SLABS):
                pltpu.make_async_copy(
                    table_hbm.at[idx_v.at[0]], bufs[b].at[g], sems[b]
                ).wait()

        start(0, 0)

        def outer(g, carry):
            for b in range(2):
                j = g * 2 + b

                @pl.when(j + 1 < nch)
                def _():
                    start(j + 1, 1 - b)

                wait(b)
                pltpu.sync_copy(
                    bufs[b], out_hbm.at[pl.ds(base + j * SLABS, SLABS)]
                )
            return carry

        lax.fori_loop(0, nch // 2, outer, 0, unroll=False)

    return grab(table, idx2d)


def kernel(arg0_1, arg1_1):
    b0, b1 = arg0_1.shape
    v, d = arg1_1.shape
    idx2d = arg0_1.astype(jnp.int32)
    return (_sc_gather(arg1_1, idx2d, b0=b0, b1=b1, d=d),)


# speedup vs baseline: 1.0129x; 1.0011x over previous
import functools

import jax
import jax.numpy as jnp
from jax import lax
from jax.experimental import pallas as pl
from jax.experimental.pallas import tpu as pltpu
from jax.experimental.pallas import tpu_sc as plsc

NC = 2
NS = 16
NW = NC * NS

SLABS = 32


def _sc_gather(table, idx2d, *, b0, b1, d):
    spw = b0 // NW
    nch = spw // SLABS
    mesh = plsc.VectorSubcoreMesh(
        core_axis_name="c", subcore_axis_name="s", num_cores=NC, num_subcores=NS
    )

    @functools.partial(
        pl.kernel,
        out_type=jax.ShapeDtypeStruct((b0, b1, d), jnp.float32),
        mesh=mesh,
        compiler_params=pltpu.CompilerParams(use_tc_tiling_on_sc=False),
        scratch_types=[
            pltpu.VMEM((spw, b1), jnp.int32),
            pltpu.VMEM((SLABS, b1, d), jnp.float32),
            pltpu.VMEM((SLABS, b1, d), jnp.float32),
            pltpu.SemaphoreType.DMA,
            pltpu.SemaphoreType.DMA,
        ],
    )
    def grab(table_hbm, idx_hbm, out_hbm, idx_v, buf0, buf1, sem0, sem1):
        wid = lax.axis_index("s") * NC + lax.axis_index("c")
        base = wid * spw
        pltpu.sync_copy(idx_hbm.at[pl.ds(base, spw)], idx_v)

        bufs = (buf0, buf1)
        sems = (sem0, sem1)

        def start(j, b):
            for g in range(SLABS):
                pltpu.make_async_copy(
                    table_hbm.at[idx_v.at[j * SLABS + g]], bufs[b].at[g], sems[b]
                ).start()

        def wait(b):
            for g in range(SLABS):
                pltpu.make_async_copy(
                    table_hbm.at[idx_v.at[0]], bufs[b].at[g], sems[b]
                ).wait()

        start(0, 0)

        def outer(g, carry):
            for b in range(2):
                j = g * 2 + b

                @pl.when(j + 1 < nch)
                def _():
                    start(j + 1, 1 - b)

                wait(b)
                pltpu.sync_copy(
                    bufs[b], out_hbm.at[pl.ds(base + j * SLABS, SLABS)]
                )
            return carry

        lax.fori_loop(0, nch // 2, outer, 0, unroll=False)

    return grab(table, idx2d)


def kernel(arg0_1, arg1_1):
    b0, b1 = arg0_1.shape
    v, d = arg1_1.shape
    idx2d = arg0_1.astype(jnp.int32)
    return (_sc_gather(arg1_1, idx2d, b0=b0, b1=b1, d=d),)
